# baseline (speedup 1.0000x reference)
import functools

import jax
import jax.numpy as jnp
from jax import lax
from jax.experimental import pallas as pl
from jax.experimental.pallas import tpu as pltpu
from jax.experimental.pallas import tpu_sc as plsc

SIGMA = 1.0

_B = 1024
_N = 200
_NC = 2
_NS = 16
_NW = _NC * _NS
_SC_ROWS = 512
_RPW = _SC_ROWS // _NW
_FPW = _RPW * _N
_NCHUNK = 13
_NP = _NCHUNK * 16
_INF = float("inf")
_TC_RB = 8


def _sc_body(sims_hbm, levs_hbm, out_hbm, ss_v, sl_v, d_v, l_v, o_v):
    wid = lax.axis_index("s") * _NC + lax.axis_index("c")
    row0 = wid * _RPW

    pltpu.sync_copy(sims_hbm.at[pl.ds(row0, _RPW)], ss_v)
    pltpu.sync_copy(levs_hbm.at[pl.ds(row0, _RPW)], sl_v)

    iota = lax.iota(jnp.int32, 16)
    inf16 = jnp.full((16,), _INF, jnp.float32)

    def lay_row(r, carry):
        dst = r * _NP
        for c in range(12):
            sv = ss_v[r, pl.ds(16 * c, 16)]
            lv = sl_v[r, pl.ds(16 * c, 16)]
            d_v[pl.ds(dst + 16 * c, 16)] = sv - SIGMA * lv
            l_v[pl.ds(dst + 16 * c, 16)] = lv
        d_v[pl.ds(dst + 192, 16)] = inf16
        l_v[pl.ds(dst + 192, 16)] = inf16
        sv = ss_v[r, pl.ds(184, 16)]
        lv = sl_v[r, pl.ds(184, 16)]
        d_v[pl.ds(dst + 184, 16)] = sv - SIGMA * lv
        l_v[pl.ds(dst + 184, 16)] = lv
        return carry
    lax.fori_loop(0, _RPW, lay_row, 0)

    zero16 = jnp.zeros((16,), jnp.float32)

    def row_body(r, accs):
        rbase = r * _NP
        dqs = [d_v[pl.ds(rbase + 16 * c, 16)] for c in range(_NCHUNK)]
        lqs = [l_v[pl.ds(rbase + 16 * c, 16)] for c in range(_NCHUNK)]

        for cp in range(_NCHUNK):
            def i_body(i, accs, cp=cp):
                pa = rbase + 16 * cp + i
                dp = jnp.full((16,), d_v[pl.ds(pa, 16)][0], jnp.float32)
                lp = jnp.full((16,), l_v[pl.ds(pa, 16)][0], jnp.float32)
                qmask = iota > jnp.full((16,), i, jnp.int32)
                new = list(accs)
                t = dp - dqs[cp]
                v = jnp.where(lp <= lqs[cp], t, -t)
                c = jnp.maximum(v, 0.0)
                new[cp] = new[cp] + jnp.where(qmask, c, zero16)
                for cq in range(cp + 1, _NCHUNK):
                    t = dp - dqs[cq]
                    v = jnp.where(lp <= lqs[cq], t, -t)
                    new[cq] = new[cq] + jnp.maximum(v, 0.0)
                return tuple(new)
            accs = lax.fori_loop(0, 16, i_body, accs)
        return accs

    accs = lax.fori_loop(0, _RPW, row_body, (zero16,) * _NCHUNK)
    total = accs[0]
    for c in range(1, _NCHUNK):
        total = total + accs[c]
    o_v[...] = total
    pltpu.sync_copy(o_v, out_hbm.at[wid])


@jax.jit
def _sc_pairwise(similarities, levs):
    mesh = plsc.VectorSubcoreMesh(core_axis_name="c", subcore_axis_name="s")
    f = functools.partial(
        pl.kernel,
        out_type=jax.ShapeDtypeStruct((_NW, 16), jnp.float32),
        mesh=mesh,
        scratch_types=[
            pltpu.VMEM((_RPW, _N), jnp.float32),
            pltpu.VMEM((_RPW, _N), jnp.float32),
            pltpu.VMEM((_RPW * _NP + 16,), jnp.float32),
            pltpu.VMEM((_RPW * _NP + 16,), jnp.float32),
            pltpu.VMEM((16,), jnp.float32),
        ],
    )(_sc_body)
    return f(similarities, levs)


def _tc_body(s_ref, l_ref, out_ref):
    s = s_ref[...]
    l = l_ref[...]
    d = s - SIGMA * l
    db = d.astype(jnp.bfloat16)
    lb = l.astype(jnp.bfloat16)
    colq = lax.broadcasted_iota(jnp.int32, (_TC_RB, _N), 1)
    colqb = colq.astype(jnp.bfloat16)
    zerob = jnp.zeros((_TC_RB, _N), jnp.bfloat16)

    acc = jnp.zeros((_TC_RB, _N), jnp.float32)
    for g in range(0, 128, 8):
        accb = zerob
        for p in range(g, g + 8):
            dp = jnp.broadcast_to(db[:, p : p + 1], (_TC_RB, _N))
            lp = jnp.broadcast_to(lb[:, p : p + 1], (_TC_RB, _N))
            t = dp - db
            v = jnp.where(lp <= lb, t, db - dp)
            c = jnp.maximum(v, jnp.bfloat16(0.0))
            accb = accb + jnp.where(colqb > jnp.bfloat16(p), c, zerob)
        acc = acc + accb.astype(jnp.float32)

    nhi = _N - 128
    dh = db[:, 128:]
    lh = lb[:, 128:]
    colqh = colqb[:, 128:]
    zeroh = jnp.zeros((_TC_RB, nhi), jnp.bfloat16)
    acch = jnp.zeros((_TC_RB, nhi), jnp.float32)
    for g in range(128, _N, 8):
        accb = zeroh
        for p in range(g, g + 8):
            dp = jnp.broadcast_to(db[:, p : p + 1], (_TC_RB, nhi))
            lp = jnp.broadcast_to(lb[:, p : p + 1], (_TC_RB, nhi))
            t = dp - dh
            v = jnp.where(lp <= lh, t, dh - dp)
            c = jnp.maximum(v, jnp.bfloat16(0.0))
            accb = accb + jnp.where(colqh > jnp.bfloat16(p), c, zeroh)
        acch = acch + accb.astype(jnp.float32)

    row_tot = jnp.sum(acc, axis=0, keepdims=True)
    row_hi = jnp.sum(acch, axis=0, keepdims=True)
    row_tot = row_tot + jnp.concatenate(
        [jnp.zeros((1, 128), jnp.float32), row_hi], axis=1)

    @pl.when(pl.program_id(0) == 0)
    def _():
        out_ref[...] = jnp.zeros_like(out_ref)

    out_ref[...] += row_tot[None]


@jax.jit
def _tc_pairwise(similarities, levs):
    nblk = (_B - _SC_ROWS) // _TC_RB
    blk0 = _SC_ROWS // _TC_RB
    return pl.pallas_call(
        _tc_body,
        out_shape=jax.ShapeDtypeStruct((1, 1, _N), jnp.float32),
        grid=(nblk,),
        in_specs=[
            pl.BlockSpec((_TC_RB, _N), lambda b: (blk0 + b, 0)),
            pl.BlockSpec((_TC_RB, _N), lambda b: (blk0 + b, 0)),
        ],
        out_specs=pl.BlockSpec((1, 1, _N), lambda b: (0, 0, 0)),
    )(similarities, levs)


def kernel(similarities, levs):
    levs = levs.reshape(similarities.shape)
    sc_part = _sc_pairwise(similarities, levs)
    tc_part = _tc_pairwise(similarities, levs)
    total = jnp.sum(sc_part) + jnp.sum(tc_part)
    return total / jnp.float32(_B * _N * _N)


# speedup vs baseline: 1.1792x; 1.1792x over previous
import functools

import jax
import jax.numpy as jnp
from jax import lax
from jax.experimental import pallas as pl
from jax.experimental.pallas import tpu as pltpu
from jax.experimental.pallas import tpu_sc as plsc

SIGMA = 1.0

_B = 1024
_N = 200
_NC = 2
_NS = 16
_NW = _NC * _NS
_SC_ROWS = 608
_RPW = _SC_ROWS // _NW
_FPW = _RPW * _N
_NCHUNK = 13
_NP = _NCHUNK * 16
_INF = float("inf")
_STG = 32
_TC_RB = 8


def _sc_body(sims_hbm, levs_hbm, out_hbm, ss_v, sl_v, d_v, l_v, o_v):
    wid = lax.axis_index("s") * _NC + lax.axis_index("c")
    row0 = wid * _RPW
    a0 = pl.multiple_of((row0 // 8) * 8, 8)
    skip = row0 - a0

    pltpu.sync_copy(sims_hbm.at[pl.ds(a0, _STG)], ss_v)
    pltpu.sync_copy(levs_hbm.at[pl.ds(a0, _STG)], sl_v)

    iota = lax.iota(jnp.int32, 16)
    inf16 = jnp.full((16,), _INF, jnp.float32)

    def lay_row(r, carry):
        rr = skip + r
        dst = r * _NP
        for c in range(12):
            sv = ss_v[rr, pl.ds(16 * c, 16)]
            lv = sl_v[rr, pl.ds(16 * c, 16)]
            d_v[pl.ds(dst + 16 * c, 16)] = sv - SIGMA * lv
            l_v[pl.ds(dst + 16 * c, 16)] = lv
        d_v[pl.ds(dst + 192, 16)] = inf16
        l_v[pl.ds(dst + 192, 16)] = inf16
        sv = ss_v[rr, pl.ds(184, 16)]
        lv = sl_v[rr, pl.ds(184, 16)]
        d_v[pl.ds(dst + 184, 16)] = sv - SIGMA * lv
        l_v[pl.ds(dst + 184, 16)] = lv
        return carry
    lax.fori_loop(0, _RPW, lay_row, 0)

    zero16 = jnp.zeros((16,), jnp.float32)

    def row_body(r, accs):
        rbase = r * _NP
        dqs = [d_v[pl.ds(rbase + 16 * c, 16)] for c in range(_NCHUNK)]
        lqs = [l_v[pl.ds(rbase + 16 * c, 16)] for c in range(_NCHUNK)]

        for cp in range(_NCHUNK):
            def i_body(i, accs, cp=cp):
                pa = rbase + 16 * cp + i
                dp = jnp.full((16,), d_v[pl.ds(pa, 16)][0], jnp.float32)
                lp = jnp.full((16,), l_v[pl.ds(pa, 16)][0], jnp.float32)
                qmask = iota > jnp.full((16,), i, jnp.int32)
                new = list(accs)
                t = dp - dqs[cp]
                v = jnp.where(lp <= lqs[cp], t, -t)
                c = jnp.maximum(v, 0.0)
                new[cp] = new[cp] + jnp.where(qmask, c, zero16)
                for cq in range(cp + 1, _NCHUNK):
                    t = dp - dqs[cq]
                    v = jnp.where(lp <= lqs[cq], t, -t)
                    new[cq] = new[cq] + jnp.maximum(v, 0.0)
                return tuple(new)
            accs = lax.fori_loop(0, 16, i_body, accs)
        return accs

    accs = lax.fori_loop(0, _RPW, row_body, (zero16,) * _NCHUNK)
    total = accs[0]
    for c in range(1, _NCHUNK):
        total = total + accs[c]
    o_v[...] = total
    pltpu.sync_copy(o_v, out_hbm.at[wid])


@jax.jit
def _sc_pairwise(similarities, levs):
    mesh = plsc.VectorSubcoreMesh(core_axis_name="c", subcore_axis_name="s")
    f = functools.partial(
        pl.kernel,
        out_type=jax.ShapeDtypeStruct((_NW, 16), jnp.float32),
        mesh=mesh,
        scratch_types=[
            pltpu.VMEM((_STG, _N), jnp.float32),
            pltpu.VMEM((_STG, _N), jnp.float32),
            pltpu.VMEM((_RPW * _NP + 16,), jnp.float32),
            pltpu.VMEM((_RPW * _NP + 16,), jnp.float32),
            pltpu.VMEM((16,), jnp.float32),
        ],
    )(_sc_body)
    return f(similarities, levs)


def _tc_body(s_ref, l_ref, out_ref):
    s = s_ref[...]
    l = l_ref[...]
    d = s - SIGMA * l
    colq = lax.broadcasted_iota(jnp.int32, (_TC_RB, _N), 1)
    zero = jnp.zeros((_TC_RB, _N), jnp.float32)

    acc = zero
    for p in range(128):
        dp = jnp.broadcast_to(d[:, p : p + 1], (_TC_RB, _N))
        lp = jnp.broadcast_to(l[:, p : p + 1], (_TC_RB, _N))
        t = dp - d
        v = jnp.where(lp <= l, t, -t)
        c = jnp.maximum(v, 0.0)
        acc = acc + jnp.where(colq > p, c, zero)

    nhi = _N - 128
    dh = d[:, 128:]
    lh = l[:, 128:]
    colqh = colq[:, 128:]
    zeroh = jnp.zeros((_TC_RB, nhi), jnp.float32)
    acch = zeroh
    for p in range(128, _N):
        dp = jnp.broadcast_to(d[:, p : p + 1], (_TC_RB, nhi))
        lp = jnp.broadcast_to(l[:, p : p + 1], (_TC_RB, nhi))
        t = dp - dh
        v = jnp.where(lp <= lh, t, -t)
        c = jnp.maximum(v, 0.0)
        acch = acch + jnp.where(colqh > p, c, zeroh)

    row_tot = jnp.sum(acc, axis=0, keepdims=True)
    row_hi = jnp.sum(acch, axis=0, keepdims=True)
    row_tot = row_tot + jnp.concatenate(
        [jnp.zeros((1, 128), jnp.float32), row_hi], axis=1)

    @pl.when(pl.program_id(0) == 0)
    def _():
        out_ref[...] = jnp.zeros_like(out_ref)

    out_ref[...] += row_tot[None]


@jax.jit
def _tc_pairwise(similarities, levs):
    nblk = (_B - _SC_ROWS) // _TC_RB
    blk0 = _SC_ROWS // _TC_RB
    return pl.pallas_call(
        _tc_body,
        out_shape=jax.ShapeDtypeStruct((1, 1, _N), jnp.float32),
        grid=(nblk,),
        in_specs=[
            pl.BlockSpec((_TC_RB, _N), lambda b: (blk0 + b, 0)),
            pl.BlockSpec((_TC_RB, _N), lambda b: (blk0 + b, 0)),
        ],
        out_specs=pl.BlockSpec((1, 1, _N), lambda b: (0, 0, 0)),
    )(similarities, levs)


def kernel(similarities, levs):
    levs = levs.reshape(similarities.shape)
    sc_part = _sc_pairwise(similarities, levs)
    tc_part = _tc_pairwise(similarities, levs)
    total = jnp.sum(sc_part) + jnp.sum(tc_part)
    return total / jnp.float32(_B * _N * _N)


# speedup vs baseline: 1.2197x; 1.0343x over previous
import functools

import jax
import jax.numpy as jnp
from jax import lax
from jax.experimental import pallas as pl
from jax.experimental.pallas import tpu as pltpu
from jax.experimental.pallas import tpu_sc as plsc

SIGMA = 1.0

_B = 1024
_N = 200
_NC = 2
_NS = 16
_NW = _NC * _NS
_SC_ROWS = 608
_RPW = _SC_ROWS // _NW
_FPW = _RPW * _N
_NCHUNK = 13
_NP = _NCHUNK * 16
_INF = float("inf")
_STG = 32
_TC_RB = 32


def _sc_body(sims_hbm, levs_hbm, out_hbm, ss_v, sl_v, d_v, l_v, o_v):
    wid = lax.axis_index("s") * _NC + lax.axis_index("c")
    row0 = wid * _RPW
    a0 = pl.multiple_of((row0 // 8) * 8, 8)
    skip = row0 - a0

    pltpu.sync_copy(sims_hbm.at[pl.ds(a0, _STG)], ss_v)
    pltpu.sync_copy(levs_hbm.at[pl.ds(a0, _STG)], sl_v)

    iota = lax.iota(jnp.int32, 16)
    inf16 = jnp.full((16,), _INF, jnp.float32)

    def lay_row(r, carry):
        rr = skip + r
        dst = r * _NP
        for c in range(12):
            sv = ss_v[rr, pl.ds(16 * c, 16)]
            lv = sl_v[rr, pl.ds(16 * c, 16)]
            d_v[pl.ds(dst + 16 * c, 16)] = sv - SIGMA * lv
            l_v[pl.ds(dst + 16 * c, 16)] = lv
        d_v[pl.ds(dst + 192, 16)] = inf16
        l_v[pl.ds(dst + 192, 16)] = inf16
        sv = ss_v[rr, pl.ds(184, 16)]
        lv = sl_v[rr, pl.ds(184, 16)]
        d_v[pl.ds(dst + 184, 16)] = sv - SIGMA * lv
        l_v[pl.ds(dst + 184, 16)] = lv
        return carry
    lax.fori_loop(0, _RPW, lay_row, 0)

    zero16 = jnp.zeros((16,), jnp.float32)

    def row_body(r, accs):
        rbase = r * _NP
        dqs = [d_v[pl.ds(rbase + 16 * c, 16)] for c in range(_NCHUNK)]
        lqs = [l_v[pl.ds(rbase + 16 * c, 16)] for c in range(_NCHUNK)]

        for cp in range(_NCHUNK):
            def i_body(i, accs, cp=cp):
                pa = rbase + 16 * cp + i
                dp = jnp.full((16,), d_v[pl.ds(pa, 16)][0], jnp.float32)
                lp = jnp.full((16,), l_v[pl.ds(pa, 16)][0], jnp.float32)
                qmask = iota > jnp.full((16,), i, jnp.int32)
                new = list(accs)
                t = dp - dqs[cp]
                v = jnp.where(lp <= lqs[cp], t, -t)
                c = jnp.maximum(v, 0.0)
                new[cp] = new[cp] + jnp.where(qmask, c, zero16)
                for cq in range(cp + 1, _NCHUNK):
                    t = dp - dqs[cq]
                    v = jnp.where(lp <= lqs[cq], t, -t)
                    new[cq] = new[cq] + jnp.maximum(v, 0.0)
                return tuple(new)
            accs = lax.fori_loop(0, 16, i_body, accs)
        return accs

    accs = lax.fori_loop(0, _RPW, row_body, (zero16,) * _NCHUNK)
    total = accs[0]
    for c in range(1, _NCHUNK):
        total = total + accs[c]
    o_v[...] = total
    pltpu.sync_copy(o_v, out_hbm.at[wid])


@jax.jit
def _sc_pairwise(similarities, levs):
    mesh = plsc.VectorSubcoreMesh(core_axis_name="c", subcore_axis_name="s")
    f = functools.partial(
        pl.kernel,
        out_type=jax.ShapeDtypeStruct((_NW, 16), jnp.float32),
        mesh=mesh,
        scratch_types=[
            pltpu.VMEM((_STG, _N), jnp.float32),
            pltpu.VMEM((_STG, _N), jnp.float32),
            pltpu.VMEM((_RPW * _NP + 16,), jnp.float32),
            pltpu.VMEM((_RPW * _NP + 16,), jnp.float32),
            pltpu.VMEM((16,), jnp.float32),
        ],
    )(_sc_body)
    return f(similarities, levs)


def _tc_body(s_ref, l_ref, out_ref):
    s = s_ref[...]
    l = l_ref[...]
    d = s - SIGMA * l
    colq = lax.broadcasted_iota(jnp.int32, (_TC_RB, _N), 1)
    zero = jnp.zeros((_TC_RB, _N), jnp.float32)

    acc = zero
    for p in range(128):
        dp = jnp.broadcast_to(d[:, p : p + 1], (_TC_RB, _N))
        lp = jnp.broadcast_to(l[:, p : p + 1], (_TC_RB, _N))
        t = dp - d
        v = jnp.where(lp <= l, t, -t)
        c = jnp.maximum(v, 0.0)
        acc = acc + jnp.where(colq > p, c, zero)

    nhi = _N - 128
    dh = d[:, 128:]
    lh = l[:, 128:]
    colqh = colq[:, 128:]
    zeroh = jnp.zeros((_TC_RB, nhi), jnp.float32)
    acch = zeroh
    for p in range(128, _N):
        dp = jnp.broadcast_to(d[:, p : p + 1], (_TC_RB, nhi))
        lp = jnp.broadcast_to(l[:, p : p + 1], (_TC_RB, nhi))
        t = dp - dh
        v = jnp.where(lp <= lh, t, -t)
        c = jnp.maximum(v, 0.0)
        acch = acch + jnp.where(colqh > p, c, zeroh)

    row_tot = jnp.sum(acc, axis=0, keepdims=True)
    row_hi = jnp.sum(acch, axis=0, keepdims=True)
    row_tot = row_tot + jnp.concatenate(
        [jnp.zeros((1, 128), jnp.float32), row_hi], axis=1)

    @pl.when(pl.program_id(0) == 0)
    def _():
        out_ref[...] = jnp.zeros_like(out_ref)

    out_ref[...] += row_tot[None]


@jax.jit
def _tc_pairwise(similarities, levs):
    nblk = (_B - _SC_ROWS) // _TC_RB
    blk0 = _SC_ROWS // _TC_RB
    return pl.pallas_call(
        _tc_body,
        out_shape=jax.ShapeDtypeStruct((1, 1, _N), jnp.float32),
        grid=(nblk,),
        in_specs=[
            pl.BlockSpec((_TC_RB, _N), lambda b: (blk0 + b, 0)),
            pl.BlockSpec((_TC_RB, _N), lambda b: (blk0 + b, 0)),
        ],
        out_specs=pl.BlockSpec((1, 1, _N), lambda b: (0, 0, 0)),
    )(similarities, levs)


def kernel(similarities, levs):
    levs = levs.reshape(similarities.shape)
    sc_part = _sc_pairwise(similarities, levs)
    tc_part = _tc_pairwise(similarities, levs)
    total = jnp.sum(sc_part) + jnp.sum(tc_part)
    return total / jnp.float32(_B * _N * _N)


# speedup vs baseline: 1.2207x; 1.0008x over previous
import functools

import jax
import jax.numpy as jnp
from jax import lax
from jax.experimental import pallas as pl
from jax.experimental.pallas import tpu as pltpu
from jax.experimental.pallas import tpu_sc as plsc

SIGMA = 1.0

_B = 1024
_N = 200
_NC = 2
_NS = 16
_NW = _NC * _NS
_SC_ROWS = 608
_RPW = _SC_ROWS // _NW
_FPW = _RPW * _N
_NCHUNK = 13
_NP = _NCHUNK * 16
_INF = float("inf")
_STG = 32
_TC_RB = 16


def _sc_body(sims_hbm, levs_hbm, out_hbm, ss_v, sl_v, d_v, l_v, o_v):
    wid = lax.axis_index("s") * _NC + lax.axis_index("c")
    row0 = wid * _RPW
    a0 = pl.multiple_of((row0 // 8) * 8, 8)
    skip = row0 - a0

    pltpu.sync_copy(sims_hbm.at[pl.ds(a0, _STG)], ss_v)
    pltpu.sync_copy(levs_hbm.at[pl.ds(a0, _STG)], sl_v)

    iota = lax.iota(jnp.int32, 16)
    inf16 = jnp.full((16,), _INF, jnp.float32)

    def lay_row(r, carry):
        rr = skip + r
        dst = r * _NP
        for c in range(12):
            sv = ss_v[rr, pl.ds(16 * c, 16)]
            lv = sl_v[rr, pl.ds(16 * c, 16)]
            d_v[pl.ds(dst + 16 * c, 16)] = sv - SIGMA * lv
            l_v[pl.ds(dst + 16 * c, 16)] = lv
        d_v[pl.ds(dst + 192, 16)] = inf16
        l_v[pl.ds(dst + 192, 16)] = inf16
        sv = ss_v[rr, pl.ds(184, 16)]
        lv = sl_v[rr, pl.ds(184, 16)]
        d_v[pl.ds(dst + 184, 16)] = sv - SIGMA * lv
        l_v[pl.ds(dst + 184, 16)] = lv
        return carry
    lax.fori_loop(0, _RPW, lay_row, 0)

    zero16 = jnp.zeros((16,), jnp.float32)

    def row_body(r, accs):
        rbase = r * _NP
        dqs = [d_v[pl.ds(rbase + 16 * c, 16)] for c in range(_NCHUNK)]
        lqs = [l_v[pl.ds(rbase + 16 * c, 16)] for c in range(_NCHUNK)]

        for cp in range(_NCHUNK):
            def i_body(i, accs, cp=cp):
                pa = rbase + 16 * cp + i
                dp = jnp.full((16,), d_v[pl.ds(pa, 16)][0], jnp.float32)
                lp = jnp.full((16,), l_v[pl.ds(pa, 16)][0], jnp.float32)
                qmask = iota > jnp.full((16,), i, jnp.int32)
                new = list(accs)
                t = dp - dqs[cp]
                v = jnp.where(lp <= lqs[cp], t, -t)
                c = jnp.maximum(v, 0.0)
                new[cp] = new[cp] + jnp.where(qmask, c, zero16)
                for cq in range(cp + 1, _NCHUNK):
                    t = dp - dqs[cq]
                    v = jnp.where(lp <= lqs[cq], t, -t)
                    new[cq] = new[cq] + jnp.maximum(v, 0.0)
                return tuple(new)
            accs = lax.fori_loop(0, 16, i_body, accs)
        return accs

    accs = lax.fori_loop(0, _RPW, row_body, (zero16,) * _NCHUNK)
    total = accs[0]
    for c in range(1, _NCHUNK):
        total = total + accs[c]
    o_v[...] = total
    pltpu.sync_copy(o_v, out_hbm.at[wid])


@jax.jit
def _sc_pairwise(similarities, levs):
    mesh = plsc.VectorSubcoreMesh(core_axis_name="c", subcore_axis_name="s")
    f = functools.partial(
        pl.kernel,
        out_type=jax.ShapeDtypeStruct((_NW, 16), jnp.float32),
        mesh=mesh,
        scratch_types=[
            pltpu.VMEM((_STG, _N), jnp.float32),
            pltpu.VMEM((_STG, _N), jnp.float32),
            pltpu.VMEM((_RPW * _NP + 16,), jnp.float32),
            pltpu.VMEM((_RPW * _NP + 16,), jnp.float32),
            pltpu.VMEM((16,), jnp.float32),
        ],
    )(_sc_body)
    return f(similarities, levs)


def _tc_body(s_ref, l_ref, out_ref):
    s = s_ref[...]
    l = l_ref[...]
    d = s - SIGMA * l
    colq = lax.broadcasted_iota(jnp.int32, (_TC_RB, _N), 1)
    zero = jnp.zeros((_TC_RB, _N), jnp.float32)

    acc = zero
    for p in range(128):
        dp = jnp.broadcast_to(d[:, p : p + 1], (_TC_RB, _N))
        lp = jnp.broadcast_to(l[:, p : p + 1], (_TC_RB, _N))
        t = dp - d
        v = jnp.where(lp <= l, t, -t)
        c = jnp.maximum(v, 0.0)
        acc = acc + jnp.where(colq > p, c, zero)

    nhi = _N - 128
    dh = d[:, 128:]
    lh = l[:, 128:]
    colqh = colq[:, 128:]
    zeroh = jnp.zeros((_TC_RB, nhi), jnp.float32)
    acch = zeroh
    for p in range(128, _N):
        dp = jnp.broadcast_to(d[:, p : p + 1], (_TC_RB, nhi))
        lp = jnp.broadcast_to(l[:, p : p + 1], (_TC_RB, nhi))
        t = dp - dh
        v = jnp.where(lp <= lh, t, -t)
        c = jnp.maximum(v, 0.0)
        acch = acch + jnp.where(colqh > p, c, zeroh)

    row_tot = jnp.sum(acc, axis=0, keepdims=True)
    row_hi = jnp.sum(acch, axis=0, keepdims=True)
    row_tot = row_tot + jnp.concatenate(
        [jnp.zeros((1, 128), jnp.float32), row_hi], axis=1)

    @pl.when(pl.program_id(0) == 0)
    def _():
        out_ref[...] = jnp.zeros_like(out_ref)

    out_ref[...] += row_tot[None]


@jax.jit
def _tc_pairwise(similarities, levs):
    nblk = (_B - _SC_ROWS) // _TC_RB
    blk0 = _SC_ROWS // _TC_RB
    return pl.pallas_call(
        _tc_body,
        out_shape=jax.ShapeDtypeStruct((1, 1, _N), jnp.float32),
        grid=(nblk,),
        in_specs=[
            pl.BlockSpec((_TC_RB, _N), lambda b: (blk0 + b, 0)),
            pl.BlockSpec((_TC_RB, _N), lambda b: (blk0 + b, 0)),
        ],
        out_specs=pl.BlockSpec((1, 1, _N), lambda b: (0, 0, 0)),
    )(similarities, levs)


def kernel(similarities, levs):
    levs = levs.reshape(similarities.shape)
    sc_part = _sc_pairwise(similarities, levs)
    tc_part = _tc_pairwise(similarities, levs)
    total = jnp.sum(sc_part) + jnp.sum(tc_part)
    return total / jnp.float32(_B * _N * _N)
